# baseline (speedup 1.0000x reference)
import functools
import jax
import jax.numpy as jnp
from jax.experimental import pallas as pl
from jax.experimental.pallas import tpu as pltpu

EPS = 1e-5


def _fused_kernel(s_steps, s_rows, g_rows, x_ref, w1_ref, b1_ref, g1_ref,
                  be1_ref, w2t_ref, b2_ref, g2_ref, be2_ref, a1_ref, a2_ref,
                  o_ref, oh_scr, hist_scr, table_scr):
    t = pl.program_id(0)

    @pl.when(t == 0)
    def _init():
        hist_scr[...] = jnp.zeros_like(hist_scr)

    @pl.when(t < s_steps)
    def _stream():
        x = x_ref[0]
        hist_scr[...] += jnp.sum(x, axis=0, keepdims=True)
        off = pl.multiple_of(t * s_rows, s_rows)
        oh_scr[pl.ds(off, s_rows), :] = x.astype(jnp.float8_e4m3fn)

    @pl.when(t == s_steps)
    def _build_table():
        inv_n = 1.0 / jnp.sum(hist_scr[...])
        cnt_row = hist_scr[...]
        hi = jnp.floor(cnt_row * (1.0 / 64.0))
        lo = cnt_row - 64.0 * hi
        stacked = jnp.concatenate([hi, lo], axis=0)
        trow = jax.lax.broadcasted_iota(jnp.int32, (2, 128), 0)
        t_w = jnp.where(trow == 0, 64, 1).astype(jnp.bfloat16)
        cnt_full = jax.lax.dot_general(
            stacked.astype(jnp.bfloat16), t_w,
            (((0,), (0,)), ((), ())),
            preferred_element_type=jnp.float32)
        cnt = cnt_full[:, 0:1]

        a1 = a1_ref[0, 0]
        a2 = a2_ref[0, 0]

        h = w1_ref[...] + b1_ref[...]
        mean1 = jnp.sum(h * cnt, axis=0, keepdims=True) * inv_n
        d = h - mean1
        var1 = jnp.sum(d * d * cnt, axis=0, keepdims=True) * inv_n
        scale1 = jax.lax.rsqrt(var1 + EPS) * g1_ref[...]
        z = d * scale1 + be1_ref[...]
        z = jnp.where(z > 0, z, a1 * z)

        y = jax.lax.dot_general(
            z, w2t_ref[...], (((1,), (1,)), ((), ())),
            preferred_element_type=jnp.float32) + b2_ref[...]
        mean2 = jnp.sum(y * cnt, axis=0, keepdims=True) * inv_n
        e = y - mean2
        var2 = jnp.sum(e * e * cnt, axis=0, keepdims=True) * inv_n
        scale2 = jax.lax.rsqrt(var2 + EPS) * g2_ref[...]
        u = e * scale2 + be2_ref[...]
        u = jnp.where(u > 0, u, a2 * u)
        hi8 = u.astype(jnp.float8_e4m3fn)
        lo8 = (u - hi8.astype(jnp.float32)).astype(jnp.float8_e4m3fn)
        table_scr[...] = jnp.concatenate([hi8, lo8], axis=1)

    @pl.when(t >= s_steps)
    def _gather():
        off = pl.multiple_of((t - s_steps) * g_rows, g_rows)
        onehot = oh_scr[pl.ds(off, g_rows), :]
        res = jax.lax.dot_general(
            table_scr[...], onehot, (((0,), (1,)), ((), ())),
            preferred_element_type=jnp.float32)
        od = o_ref.shape[1]
        o_ref[0] = res[:od, :] + res[od:, :]


def kernel(pos_onehot, w1, b1, g1, be1, a1, w2, b2, g2, be2, a2):
    b, l, classes = pos_onehot.shape
    out_dim = w2.shape[1]
    n = b * l

    split = 2
    s_steps = b * split
    s_rows = l // split
    g_steps = b
    g_rows = l

    w2t = jnp.swapaxes(w2, 0, 1)

    const = lambda shape: pl.BlockSpec(shape, lambda i, s=len(shape): (0,) * s)
    smem = pl.BlockSpec(memory_space=pltpu.MemorySpace.SMEM)

    body = functools.partial(_fused_kernel, s_steps, s_rows, g_rows)

    def x_map(t):
        tt = jnp.minimum(t, s_steps - 1)
        return (tt // split, tt % split, 0)

    out_t = pl.pallas_call(
        body,
        out_shape=jax.ShapeDtypeStruct((b, out_dim, l), jnp.float32),
        grid=(s_steps + g_steps,),
        in_specs=[
            pl.BlockSpec((1, s_rows, classes), x_map),
            const(w1.shape),
            const(b1.shape), const(g1.shape), const(be1.shape),
            const(w2t.shape), const(b2.shape), const(g2.shape),
            const(be2.shape), smem, smem,
        ],
        out_specs=pl.BlockSpec((1, out_dim, g_rows),
                               lambda t: (jnp.maximum(t - s_steps, 0), 0, 0)),
        scratch_shapes=[pltpu.VMEM((n, classes), jnp.float8_e4m3fn),
                        pltpu.VMEM((1, classes), jnp.float32),
                        pltpu.VMEM((classes, 2 * out_dim), jnp.float8_e4m3fn)],
        compiler_params=pltpu.CompilerParams(
            dimension_semantics=("arbitrary",)),
    )(pos_onehot, w1, b1, g1, be1, w2t, b2, g2, be2, a1, a2)

    return jnp.swapaxes(out_t, 1, 2)


# speedup vs baseline: 1.0673x; 1.0673x over previous
import functools
import jax
import jax.numpy as jnp
from jax.experimental import pallas as pl
from jax.experimental.pallas import tpu as pltpu

EPS = 1e-5


def _fused_kernel(s_steps, s_rows, g_rows, x_ref, w1_ref, b1_ref, g1_ref,
                  be1_ref, w2t_ref, b2_ref, g2_ref, be2_ref, a1_ref, a2_ref,
                  o_ref, oh_scr, hist_scr, table_scr):
    t = pl.program_id(0)

    @pl.when(t == 0)
    def _init():
        hist_scr[...] = jnp.zeros_like(hist_scr)

    @pl.when(t < s_steps)
    def _stream():
        x = x_ref[0]
        hist_scr[...] += jnp.sum(x, axis=0, keepdims=True)
        off = pl.multiple_of(t * s_rows, s_rows)
        oh_scr[pl.ds(off, s_rows), :] = x.astype(jnp.float8_e4m3fn)

    @pl.when(t == s_steps)
    def _build_table():
        inv_n = 1.0 / jnp.sum(hist_scr[...])
        cnt_row = hist_scr[...]
        hi = jnp.floor(cnt_row * (1.0 / 64.0))
        lo = cnt_row - 64.0 * hi
        stacked = jnp.concatenate([hi, lo], axis=0)
        trow = jax.lax.broadcasted_iota(jnp.int32, (2, 128), 0)
        t_w = jnp.where(trow == 0, 64, 1).astype(jnp.bfloat16)
        cnt_full = jax.lax.dot_general(
            stacked.astype(jnp.bfloat16), t_w,
            (((0,), (0,)), ((), ())),
            preferred_element_type=jnp.float32)
        cnt = cnt_full[:, 0:1]

        a1 = a1_ref[0, 0]
        a2 = a2_ref[0, 0]

        h = w1_ref[...] + b1_ref[...]
        mean1 = jnp.sum(h * cnt, axis=0, keepdims=True) * inv_n
        d = h - mean1
        var1 = jnp.sum(d * d * cnt, axis=0, keepdims=True) * inv_n
        scale1 = jax.lax.rsqrt(var1 + EPS) * g1_ref[...]
        z = d * scale1 + be1_ref[...]
        z = jnp.where(z > 0, z, a1 * z)

        y = jax.lax.dot_general(
            z, w2t_ref[...], (((1,), (1,)), ((), ())),
            preferred_element_type=jnp.float32) + b2_ref[...]
        mean2 = jnp.sum(y * cnt, axis=0, keepdims=True) * inv_n
        e = y - mean2
        var2 = jnp.sum(e * e * cnt, axis=0, keepdims=True) * inv_n
        scale2 = jax.lax.rsqrt(var2 + EPS) * g2_ref[...]
        u = e * scale2 + be2_ref[...]
        u = jnp.where(u > 0, u, a2 * u)
        hi8 = u.astype(jnp.float8_e4m3fn)
        lo8 = (u - hi8.astype(jnp.float32)).astype(jnp.float8_e4m3fn)
        table_scr[...] = jnp.concatenate([hi8, lo8], axis=1)

    @pl.when(t >= s_steps)
    def _gather():
        off = pl.multiple_of((t - s_steps) * g_rows, g_rows)
        onehot = oh_scr[pl.ds(off, g_rows), :]
        res = jax.lax.dot_general(
            table_scr[...], onehot, (((0,), (1,)), ((), ())),
            preferred_element_type=jnp.float32)
        od = o_ref.shape[1]
        o_ref[0] = res[:od, :] + res[od:, :]


def kernel(pos_onehot, w1, b1, g1, be1, a1, w2, b2, g2, be2, a2):
    b, l, classes = pos_onehot.shape
    out_dim = w2.shape[1]
    n = b * l

    s_steps = b
    s_rows = l
    g_steps = b
    g_rows = l

    w2t = jnp.swapaxes(w2, 0, 1)

    const = lambda shape: pl.BlockSpec(shape, lambda i, s=len(shape): (0,) * s)
    smem = pl.BlockSpec(memory_space=pltpu.MemorySpace.SMEM)

    body = functools.partial(_fused_kernel, s_steps, s_rows, g_rows)

    def x_map(t):
        tt = jnp.minimum(t, s_steps - 1)
        return (tt, 0, 0)

    out_t = pl.pallas_call(
        body,
        out_shape=jax.ShapeDtypeStruct((b, out_dim, l), jnp.float32),
        grid=(s_steps + g_steps,),
        in_specs=[
            pl.BlockSpec((1, s_rows, classes), x_map),
            const(w1.shape),
            const(b1.shape), const(g1.shape), const(be1.shape),
            const(w2t.shape), const(b2.shape), const(g2.shape),
            const(be2.shape), smem, smem,
        ],
        out_specs=pl.BlockSpec((1, out_dim, g_rows),
                               lambda t: (jnp.maximum(t - s_steps, 0), 0, 0)),
        scratch_shapes=[pltpu.VMEM((n, classes), jnp.float8_e4m3fn),
                        pltpu.VMEM((1, classes), jnp.float32),
                        pltpu.VMEM((classes, 2 * out_dim), jnp.float8_e4m3fn)],
        compiler_params=pltpu.CompilerParams(
            dimension_semantics=("arbitrary",)),
    )(pos_onehot, w1, b1, g1, be1, w2t, b2, g2, be2, a1, a2)

    return jnp.swapaxes(out_t, 1, 2)


# speedup vs baseline: 1.0781x; 1.0101x over previous
import functools
import jax
import jax.numpy as jnp
from jax.experimental import pallas as pl
from jax.experimental.pallas import tpu as pltpu

EPS = 1e-5


def _fused_kernel(s_steps, s_rows, g_rows, x_ref, w1_ref, b1_ref, g1_ref,
                  be1_ref, w2t_ref, b2_ref, g2_ref, be2_ref, a1_ref, a2_ref,
                  o_ref, oh_scr, hist_scr, table_scr):
    t = pl.program_id(0)

    @pl.when(t == 0)
    def _init():
        hist_scr[...] = jnp.zeros_like(hist_scr)

    @pl.when(t < s_steps)
    def _stream():
        x = x_ref[0]
        hist_scr[...] += jnp.sum(x, axis=0, keepdims=True)
        off = pl.multiple_of(t * s_rows, s_rows)
        oh_scr[pl.ds(off, s_rows), :] = x.astype(jnp.float8_e4m3fn)

    @pl.when(t == s_steps)
    def _build_table():
        inv_n = 1.0 / jnp.sum(hist_scr[...])
        cnt_row = hist_scr[...]
        hi = jnp.floor(cnt_row * (1.0 / 64.0))
        lo = cnt_row - 64.0 * hi
        stacked = jnp.concatenate([hi, lo], axis=0)
        trow = jax.lax.broadcasted_iota(jnp.int32, (2, 128), 0)
        t_w = jnp.where(trow == 0, 64, 1).astype(jnp.bfloat16)
        cnt_full = jax.lax.dot_general(
            stacked.astype(jnp.bfloat16), t_w,
            (((0,), (0,)), ((), ())),
            preferred_element_type=jnp.float32)
        cnt = cnt_full[:, 0:1]

        a1 = a1_ref[0, 0]
        a2 = a2_ref[0, 0]

        h = w1_ref[...] + b1_ref[...]
        mean1 = jnp.sum(h * cnt, axis=0, keepdims=True) * inv_n
        d = h - mean1
        var1 = jnp.sum(d * d * cnt, axis=0, keepdims=True) * inv_n
        scale1 = jax.lax.rsqrt(var1 + EPS) * g1_ref[...]
        z = d * scale1 + be1_ref[...]
        z = jnp.where(z > 0, z, a1 * z)

        y = jax.lax.dot_general(
            z, w2t_ref[...], (((1,), (1,)), ((), ())),
            preferred_element_type=jnp.float32) + b2_ref[...]
        mean2 = jnp.sum(y * cnt, axis=0, keepdims=True) * inv_n
        e = y - mean2
        var2 = jnp.sum(e * e * cnt, axis=0, keepdims=True) * inv_n
        scale2 = jax.lax.rsqrt(var2 + EPS) * g2_ref[...]
        u = e * scale2 + be2_ref[...]
        u = jnp.where(u > 0, u, a2 * u)
        hi8 = u.astype(jnp.float8_e4m3fn)
        lo8 = (u - hi8.astype(jnp.float32)).astype(jnp.float8_e4m3fn)
        table_scr[...] = jnp.concatenate([hi8, lo8], axis=1)

    @pl.when(t >= s_steps)
    def _gather():
        res = jax.lax.dot_general(
            table_scr[...], oh_scr[...], (((0,), (1,)), ((), ())),
            preferred_element_type=jnp.float32)
        od = o_ref.shape[1]
        for i in range(o_ref.shape[0]):
            lo_c, hi_c = i * g_rows, (i + 1) * g_rows
            o_ref[i] = res[:od, lo_c:hi_c] + res[od:, lo_c:hi_c]


def kernel(pos_onehot, w1, b1, g1, be1, a1, w2, b2, g2, be2, a2):
    b, l, classes = pos_onehot.shape
    out_dim = w2.shape[1]
    n = b * l

    s_steps = b
    s_rows = l
    g_steps = 1
    g_rows = l

    w2t = jnp.swapaxes(w2, 0, 1)

    const = lambda shape: pl.BlockSpec(shape, lambda i, s=len(shape): (0,) * s)
    smem = pl.BlockSpec(memory_space=pltpu.MemorySpace.SMEM)

    body = functools.partial(_fused_kernel, s_steps, s_rows, g_rows)

    def x_map(t):
        tt = jnp.minimum(t, s_steps - 1)
        return (tt, 0, 0)

    out_t = pl.pallas_call(
        body,
        out_shape=jax.ShapeDtypeStruct((b, out_dim, l), jnp.float32),
        grid=(s_steps + g_steps,),
        in_specs=[
            pl.BlockSpec((1, s_rows, classes), x_map),
            const(w1.shape),
            const(b1.shape), const(g1.shape), const(be1.shape),
            const(w2t.shape), const(b2.shape), const(g2.shape),
            const(be2.shape), smem, smem,
        ],
        out_specs=pl.BlockSpec((b, out_dim, g_rows), lambda t: (0, 0, 0)),
        scratch_shapes=[pltpu.VMEM((n, classes), jnp.float8_e4m3fn),
                        pltpu.VMEM((1, classes), jnp.float32),
                        pltpu.VMEM((classes, 2 * out_dim), jnp.float8_e4m3fn)],
        compiler_params=pltpu.CompilerParams(
            dimension_semantics=("arbitrary",)),
    )(pos_onehot, w1, b1, g1, be1, w2t, b2, g2, be2, a1, a2)

    return jnp.swapaxes(out_t, 1, 2)
